# Initial kernel scaffold; baseline (speedup 1.0000x reference)
import jax
import jax.numpy as jnp
from jax.experimental import pallas as pl


def kernel(x, packed_weights, scales, zero_points):
    raise NotImplementedError("write your pallas kernel here")


# speedup vs baseline: 1.6713x; 1.6713x over previous
import jax
import jax.numpy as jnp
from jax.experimental import pallas as pl
from jax.experimental.pallas import tpu as pltpu

_BM = 1024
_BN = 512


def _qlin_kernel(xe_ref, xo_ref, pk_ref, s_ref, zp_ref, o_ref, xsum_ref):
    n = pl.program_id(1)

    @pl.when(n == 0)
    def _():
        xsum_ref[...] = (
            jnp.sum(xe_ref[...].astype(jnp.float32), axis=1, keepdims=True)
            + jnp.sum(xo_ref[...].astype(jnp.float32), axis=1, keepdims=True)
        )

    p = pk_ref[...].astype(jnp.int32)
    q_low = (p & 15).astype(jnp.bfloat16)
    q_high = ((p >> 4) & 15).astype(jnp.bfloat16)
    acc = jnp.dot(xe_ref[...], q_low, preferred_element_type=jnp.float32)
    acc = acc + jnp.dot(xo_ref[...], q_high, preferred_element_type=jnp.float32)
    s = s_ref[...]
    zp = zp_ref[...]
    o_ref[...] = acc * s - xsum_ref[...] * (s * zp)


@jax.jit
def kernel(x, packed_weights, scales, zero_points):
    B, IN_F = x.shape
    OUT_F = packed_weights.shape[0]
    Kp = IN_F // 2

    xr = x.reshape(B, Kp, 2).astype(jnp.bfloat16)
    xe = xr[:, :, 0]
    xo = xr[:, :, 1]
    pk = packed_weights.astype(jnp.uint8).T
    s2 = scales.reshape(1, OUT_F)
    zp2 = zero_points.reshape(1, OUT_F)

    grid = (B // _BM, pl.cdiv(OUT_F, _BN))
    return pl.pallas_call(
        _qlin_kernel,
        out_shape=jax.ShapeDtypeStruct((B, OUT_F), jnp.float32),
        grid=grid,
        in_specs=[
            pl.BlockSpec((_BM, Kp), lambda m, n: (m, 0)),
            pl.BlockSpec((_BM, Kp), lambda m, n: (m, 0)),
            pl.BlockSpec((Kp, _BN), lambda m, n: (0, n)),
            pl.BlockSpec((1, _BN), lambda m, n: (0, n)),
            pl.BlockSpec((1, _BN), lambda m, n: (0, n)),
        ],
        out_specs=pl.BlockSpec((_BM, _BN), lambda m, n: (m, n)),
        scratch_shapes=[pltpu.VMEM((_BM, 1), jnp.float32)],
        compiler_params=pltpu.CompilerParams(
            dimension_semantics=("parallel", "arbitrary"),
        ),
        name="qlin_int4",
    )(xe, xo, pk, s2, zp2)


# speedup vs baseline: 2.1818x; 1.3054x over previous
import jax
import jax.numpy as jnp
from jax.experimental import pallas as pl
from jax.experimental.pallas import tpu as pltpu

_BM = 1024
_BN = 512


_DN = (((1,), (1,)), ((), ()))


def _qlin_kernel(xeo_ref, pk_ref, s_ref, zp_ref, o_ref, xsum_ref):
    n = pl.program_id(1)
    xe = xeo_ref[0]
    xo = xeo_ref[1]

    @pl.when(n == 0)
    def _():
        xsum_ref[...] = (
            jnp.sum(xe.astype(jnp.float32), axis=1, keepdims=True)
            + jnp.sum(xo.astype(jnp.float32), axis=1, keepdims=True)
        )

    p = pk_ref[...].astype(jnp.int32)
    q_low = (p & 15).astype(jnp.bfloat16)
    q_high = ((p >> 4) & 15).astype(jnp.bfloat16)
    acc = jax.lax.dot_general(xe, q_low, _DN, preferred_element_type=jnp.float32)
    acc = acc + jax.lax.dot_general(xo, q_high, _DN, preferred_element_type=jnp.float32)
    s = s_ref[...]
    zp = zp_ref[...]
    o_ref[...] = acc * s - xsum_ref[...] * (s * zp)


@jax.jit
def kernel(x, packed_weights, scales, zero_points):
    B, IN_F = x.shape
    OUT_F = packed_weights.shape[0]
    Kp = IN_F // 2

    xeo = x.reshape(B, Kp, 2).transpose(2, 0, 1).astype(jnp.bfloat16)
    pk = packed_weights.astype(jnp.uint8)
    s2 = scales.reshape(1, OUT_F)
    zp2 = zero_points.reshape(1, OUT_F)

    grid = (B // _BM, pl.cdiv(OUT_F, _BN))
    return pl.pallas_call(
        _qlin_kernel,
        out_shape=jax.ShapeDtypeStruct((B, OUT_F), jnp.float32),
        grid=grid,
        in_specs=[
            pl.BlockSpec((2, _BM, Kp), lambda m, n: (0, m, 0)),
            pl.BlockSpec((_BN, Kp), lambda m, n: (n, 0)),
            pl.BlockSpec((1, _BN), lambda m, n: (0, n)),
            pl.BlockSpec((1, _BN), lambda m, n: (0, n)),
        ],
        out_specs=pl.BlockSpec((_BM, _BN), lambda m, n: (m, n)),
        scratch_shapes=[pltpu.VMEM((_BM, 1), jnp.float32)],
        compiler_params=pltpu.CompilerParams(
            dimension_semantics=("parallel", "arbitrary"),
        ),
        name="qlin_int4",
    )(xeo, pk, s2, zp2)


# speedup vs baseline: 2.5324x; 1.1607x over previous
import jax
import jax.numpy as jnp
from jax.experimental import pallas as pl
from jax.experimental.pallas import tpu as pltpu

_BM = 1024
_BN = 512
_DN = (((1,), (1,)), ((), ()))


def _qlin_kernel(x_ref, pk_ref, s_ref, zp_ref, o_ref, xsum_ref):
    n = pl.program_id(1)

    @pl.when(n == 0)
    def _():
        xsum_ref[...] = jnp.sum(
            x_ref[...].astype(jnp.float32), axis=1, keepdims=True
        )

    p = pk_ref[...].astype(jnp.int32)
    w32 = (p & 15) | ((p & 0xF0) << 12) | 0x43004300
    q = pltpu.bitcast(w32, jnp.bfloat16)
    acc = jnp.dot(x_ref[...], q, preferred_element_type=jnp.float32)
    s = s_ref[...]
    zpb = zp_ref[...] + jnp.float32(128.0)
    o_ref[...] = acc * s - xsum_ref[...] * (s * zpb)


@jax.jit
def kernel(x, packed_weights, scales, zero_points):
    B, IN_F = x.shape
    OUT_F = packed_weights.shape[0]
    Kp = IN_F // 2

    xde = x.astype(jnp.bfloat16)
    pk = packed_weights.astype(jnp.uint8).T
    s2 = scales.reshape(1, OUT_F)
    zp2 = zero_points.reshape(1, OUT_F)

    grid = (B // _BM, pl.cdiv(OUT_F, _BN))
    return pl.pallas_call(
        _qlin_kernel,
        out_shape=jax.ShapeDtypeStruct((B, OUT_F), jnp.float32),
        grid=grid,
        in_specs=[
            pl.BlockSpec((_BM, IN_F), lambda m, n: (m, 0)),
            pl.BlockSpec((Kp, _BN), lambda m, n: (0, n)),
            pl.BlockSpec((1, _BN), lambda m, n: (0, n)),
            pl.BlockSpec((1, _BN), lambda m, n: (0, n)),
        ],
        out_specs=pl.BlockSpec((_BM, _BN), lambda m, n: (m, n)),
        scratch_shapes=[pltpu.VMEM((_BM, 1), jnp.float32)],
        compiler_params=pltpu.CompilerParams(
            dimension_semantics=("arbitrary", "arbitrary"),
        ),
        name="qlin_int4",
    )(xde, pk, s2, zp2)
